# Initial kernel scaffold; baseline (speedup 1.0000x reference)
import jax
import jax.numpy as jnp
from jax.experimental import pallas as pl


def kernel(x, edge_index, W1, b1, W2, b2, Wm1, bm1, Wm2, bm2):
    raise NotImplementedError("write your pallas kernel here")


# speedup vs baseline: 2.8050x; 2.8050x over previous
import functools

import jax
import jax.numpy as jnp
from jax import lax
from jax.experimental import pallas as pl
from jax.experimental.pallas import tpu as pltpu
from jax.experimental.pallas import tpu_sc as plsc

N = 10000
E = 320000
F = 128
H = 153
HP = 160
CB = 128
HI = HP - CB
NCLS = 2

NC, NS = 2, 16
EPT = E // NS
CE = 80
NCH = EPT // CE
NP = 10240
RP = NP // NS
RC = 128
DW = 128
EPW = E // (NC * NS)

TH1 = (-0.018739098133068916, 0.22664318420656426,
       -1.1625027523916962, 1.3784681394089935)
TH2 = (0.11613730625866586, -0.9204508026677373,
       2.2984110493405274, -0.8451376850831508)


def _deg_body(dst_hbm, ones_hbm, zeros_hbm, out_hbm,
              dst_v, ones_v, stage_v, deg_sh):
    c = lax.axis_index("c")
    s = lax.axis_index("s")
    wid = c * NS + s
    pltpu.sync_copy(ones_hbm, ones_v)
    pltpu.sync_copy(zeros_hbm, stage_v)

    def zinit(j, carry):
        pltpu.sync_copy(stage_v, deg_sh.at[pl.ds(s * RP + j * RC, RC), :])
        return carry

    lax.fori_loop(0, RP // RC, zinit, 0)
    plsc.subcore_barrier()

    def body(i, carry):
        off = pl.multiple_of(wid * EPW + i * CE, 8)
        pltpu.sync_copy(dst_hbm.at[pl.ds(off, CE)], dst_v)
        pltpu.sync_copy(ones_v, deg_sh.at[dst_v], add=True)
        return carry

    lax.fori_loop(0, EPW // CE, body, 0)
    plsc.subcore_barrier()

    def cout(j, carry):
        r = s * RP + j * RC
        pltpu.sync_copy(deg_sh.at[pl.ds(r, RC), :], stage_v)
        pltpu.sync_copy(stage_v, out_hbm.at[c, pl.ds(r, RC), :])
        return carry

    lax.fori_loop(0, RP // RC, cout, 0)


def _scatter_body(ylo_hbm, yhi_hbm, src_hbm, dst_hbm, zeros_hbm, out_hbm,
                  src_v, dst_v, rows_v, stage_v, agg_sh, sem):
    c = lax.axis_index("c")
    s = lax.axis_index("s")
    pltpu.sync_copy(zeros_hbm, stage_v)

    def zinit(j, carry):
        pltpu.sync_copy(stage_v, agg_sh.at[pl.ds(s * RP + j * RC, RC), :])
        return carry

    lax.fori_loop(0, RP // RC, zinit, 0)
    plsc.subcore_barrier()

    def edge_loop(tbl_hbm):
        def body(i, carry):
            off = pl.multiple_of(s * EPT + i * CE, 8)
            pltpu.sync_copy(src_hbm.at[pl.ds(off, CE)], src_v)
            pltpu.sync_copy(dst_hbm.at[pl.ds(off, CE)], dst_v)
            pltpu.async_copy(tbl_hbm.at[src_v], rows_v, sem).wait()
            pltpu.sync_copy(rows_v, agg_sh.at[dst_v], add=True)
            return carry

        lax.fori_loop(0, NCH, body, 0)

    pl.when(c == 0)(lambda: edge_loop(ylo_hbm))
    pl.when(c == 1)(lambda: edge_loop(yhi_hbm))
    plsc.subcore_barrier()

    def cout(j, carry):
        r = s * RP + j * RC
        pltpu.sync_copy(agg_sh.at[pl.ds(r, RC), :], stage_v)
        pltpu.sync_copy(stage_v, out_hbm.at[c, pl.ds(r, RC), :])
        return carry

    lax.fori_loop(0, RP // RC, cout, 0)


@functools.cache
def _build_sc_kernels():
    mesh = plsc.VectorSubcoreMesh(core_axis_name="c", subcore_axis_name="s",
                                  num_cores=NC, num_subcores=NS)
    deg = pl.kernel(
        _deg_body,
        out_type=jax.ShapeDtypeStruct((NC, NP, DW), jnp.float32),
        mesh=mesh,
        scratch_types=[
            pltpu.VMEM((CE,), jnp.int32),
            pltpu.VMEM((CE, DW), jnp.float32),
            pltpu.VMEM((RC, DW), jnp.float32),
            pltpu.VMEM_SHARED((NP, DW), jnp.float32),
        ],
    )
    scat = pl.kernel(
        _scatter_body,
        out_type=jax.ShapeDtypeStruct((NC, NP, CB), jnp.float32),
        mesh=mesh,
        scratch_types=[
            pltpu.VMEM((CE,), jnp.int32),
            pltpu.VMEM((CE,), jnp.int32),
            pltpu.VMEM((CE, CB), jnp.float32),
            pltpu.VMEM((RC, CB), jnp.float32),
            pltpu.VMEM_SHARED((NP, CB), jnp.float32),
            pltpu.SemaphoreType.DMA,
        ],
    )
    return deg, scat


def _deg_kernel(dst, ones_dw, zeros_dw):
    return _build_sc_kernels()[0](dst, ones_dw, zeros_dw)


def _scatter_kernel(ylo, yhi, src, dst, zeros_cb):
    return _build_sc_kernels()[1](ylo, yhi, src, dst, zeros_cb)


_RB = 1000
_GRID = (N // _RB,)


def _row_spec(w):
    return pl.BlockSpec((_RB, w), lambda i: (i, 0))


def _part_spec(part, w):
    return pl.BlockSpec((1, _RB, w), lambda i, _p=part: (_p, i, 0))


def _full_spec(r, ccols):
    return pl.BlockSpec((r, ccols), lambda i: (0, 0))


def _mlp_body(x_ref, w1_ref, b1_ref, w2_ref, b2_ref, h_ref):
    h1 = jnp.dot(x_ref[...], w1_ref[...], preferred_element_type=jnp.float32)
    h1 = jnp.maximum(h1 + b1_ref[...], 0.0)
    h2 = jnp.dot(h1, w2_ref[...], preferred_element_type=jnp.float32)
    h_ref[...] = jnp.maximum(h2 + b2_ref[...], 0.0)


def _mlp_call(x, w1p, b1p, w2p, b2p):
    return pl.pallas_call(
        _mlp_body,
        grid=_GRID,
        in_specs=[_row_spec(F), _full_spec(F, HP), _full_spec(1, HP),
                  _full_spec(HP, HP), _full_spec(1, HP)],
        out_specs=_row_spec(HP),
        out_shape=jax.ShapeDtypeStruct((N, HP), jnp.float32),
    )(x, w1p, b1p, w2p, b2p)


def _split_y(fn, d, ylo_ref, yhi_ref):
    y = fn * d
    ylo_ref[...] = y[:, :CB]
    yhi_ref[...] = jnp.concatenate(
        [y[:, CB:], jnp.zeros((y.shape[0], CB - HI), jnp.float32)], axis=1)


def _dinv_body(d0_ref, d1_ref, h_ref, dinv_ref, ylo_ref, yhi_ref,
               acc1_ref, acc2_ref):
    deg = d0_ref[0] + d1_ref[0]
    dinv = lax.rsqrt(jnp.maximum(deg, 1.0))
    dinv_ref[...] = dinv
    h = h_ref[...]
    _split_y(h, dinv[:, 0:1], ylo_ref, yhi_ref)
    acc1_ref[...] = TH1[0] * h
    acc2_ref[...] = TH2[0] * h


def _dinv_call(degp, h):
    return pl.pallas_call(
        _dinv_body,
        grid=_GRID,
        in_specs=[_part_spec(0, DW), _part_spec(1, DW), _row_spec(HP)],
        out_specs=[_row_spec(DW), _row_spec(CB), _row_spec(CB),
                   _row_spec(HP), _row_spec(HP)],
        out_shape=[jax.ShapeDtypeStruct((N, DW), jnp.float32),
                   jax.ShapeDtypeStruct((N, CB), jnp.float32),
                   jax.ShapeDtypeStruct((N, CB), jnp.float32),
                   jax.ShapeDtypeStruct((N, HP), jnp.float32),
                   jax.ShapeDtypeStruct((N, HP), jnp.float32)],
    )(degp, degp, h)


def _combine_body(th_ref, feat_ref, a0_ref, a1_ref, dinv_ref, acc1_ref,
                  acc2_ref, featn_ref, ylo_ref, yhi_ref,
                  acc1o_ref, acc2o_ref):
    d = dinv_ref[:, 0:1]
    agg = jnp.concatenate([a0_ref[0], a1_ref[0][:, :HI]], axis=1)
    fn = feat_ref[...] - agg * d
    featn_ref[...] = fn
    _split_y(fn, d, ylo_ref, yhi_ref)
    acc1o_ref[...] = acc1_ref[...] + th_ref[0] * fn
    acc2o_ref[...] = acc2_ref[...] + th_ref[1] * fn


def _combine_call(th, feat, aggp, dinv, acc1, acc2):
    return pl.pallas_call(
        _combine_body,
        grid=_GRID,
        in_specs=[pl.BlockSpec(memory_space=pltpu.SMEM),
                  _row_spec(HP), _part_spec(0, CB), _part_spec(1, CB),
                  _row_spec(DW), _row_spec(HP), _row_spec(HP)],
        out_specs=[_row_spec(HP), _row_spec(CB), _row_spec(CB),
                   _row_spec(HP), _row_spec(HP)],
        out_shape=[jax.ShapeDtypeStruct((N, HP), jnp.float32),
                   jax.ShapeDtypeStruct((N, CB), jnp.float32),
                   jax.ShapeDtypeStruct((N, CB), jnp.float32),
                   jax.ShapeDtypeStruct((N, HP), jnp.float32),
                   jax.ShapeDtypeStruct((N, HP), jnp.float32)],
    )(th, feat, aggp, aggp, dinv, acc1, acc2)


def _head_body(h_ref, acc1_ref, acc2_ref, wa_ref, wb_ref, wc_ref,
               bm1_ref, wm2_ref, bm2_ref, out_ref):
    z = jnp.dot(h_ref[...], wa_ref[...], preferred_element_type=jnp.float32)
    z += jnp.dot(acc1_ref[...], wb_ref[...], preferred_element_type=jnp.float32)
    z += jnp.dot(acc2_ref[...], wc_ref[...], preferred_element_type=jnp.float32)
    z = jnp.maximum(z + bm1_ref[...], 0.0)
    out_ref[...] = (jnp.dot(z, wm2_ref[...], preferred_element_type=jnp.float32)
                    + bm2_ref[...])


def _head_call(h, acc1, acc2, wa, wb, wc, bm1p, wm2p, bm2):
    return pl.pallas_call(
        _head_body,
        grid=_GRID,
        in_specs=[_row_spec(HP)] * 3 + [_full_spec(HP, HP)] * 3
                 + [_full_spec(1, HP), _full_spec(HP, NCLS), _full_spec(1, NCLS)],
        out_specs=_row_spec(NCLS),
        out_shape=jax.ShapeDtypeStruct((N, NCLS), jnp.float32),
    )(h, acc1, acc2, wa, wb, wc, bm1p, wm2p, bm2)


def kernel(x, edge_index, W1, b1, W2, b2, Wm1, bm1, Wm2, bm2):
    src = edge_index[0]
    dst = edge_index[1]

    w1p = jnp.pad(W1, ((0, 0), (0, HP - H)))
    b1p = jnp.pad(b1, (0, HP - H)).reshape(1, HP)
    w2p = jnp.pad(W2, ((0, HP - H), (0, HP - H)))
    b2p = jnp.pad(b2, (0, HP - H)).reshape(1, HP)
    wa = jnp.pad(Wm1[0:H], ((0, HP - H), (0, HP - H)))
    wb = jnp.pad(Wm1[H:2 * H], ((0, HP - H), (0, HP - H)))
    wc = jnp.pad(Wm1[2 * H:3 * H], ((0, HP - H), (0, HP - H)))
    bm1p = jnp.pad(bm1, (0, HP - H)).reshape(1, HP)
    wm2p = jnp.pad(Wm2, ((0, HP - H), (0, 0)))
    bm2p = bm2.reshape(1, NCLS)

    ones_dw = jnp.ones((CE, DW), jnp.float32)
    zeros_dw = jnp.zeros((RC, DW), jnp.float32)
    zeros_cb = jnp.zeros((RC, CB), jnp.float32)

    h = _mlp_call(x, w1p, b1p, w2p, b2p)
    degp = _deg_kernel(dst, ones_dw, zeros_dw)
    dinv, ylo, yhi, acc1, acc2 = _dinv_call(degp, h)

    ths = jnp.array([[TH1[1], TH2[1]], [TH1[2], TH2[2]], [TH1[3], TH2[3]]],
                    jnp.float32)

    def step(carry, th):
        feat, ylo, yhi, acc1, acc2 = carry
        aggp = _scatter_kernel(ylo, yhi, src, dst, zeros_cb)
        feat, ylo, yhi, acc1, acc2 = _combine_call(th, feat, aggp, dinv,
                                                   acc1, acc2)
        return (feat, ylo, yhi, acc1, acc2), None

    (_, _, _, acc1, acc2), _ = lax.scan(step, (h, ylo, yhi, acc1, acc2), ths)

    return _head_call(h, acc1, acc2, wa, wb, wc, bm1p, wm2p, bm2p)


# speedup vs baseline: 4.4533x; 1.5876x over previous
import functools

import jax
import jax.numpy as jnp
from jax import lax
from jax.experimental import pallas as pl
from jax.experimental.pallas import tpu as pltpu
from jax.experimental.pallas import tpu_sc as plsc

N = 10000
E = 320000
F = 128
H = 153
HP = 160
CB = 128
HI = HP - CB
NCLS = 2

NC, NS = 2, 16
EPT = E // NS
CE = 80
NCH = EPT // CE
NP = 10240
RP = NP // NS
RC = 64
DW = 128
EPW = E // (NC * NS)

TH1 = (-0.018739098133068916, 0.22664318420656426,
       -1.1625027523916962, 1.3784681394089935)
TH2 = (0.11613730625866586, -0.9204508026677373,
       2.2984110493405274, -0.8451376850831508)


def _deg_body(dst_hbm, ones_hbm, zeros_hbm, out_hbm,
              dst_v, ones_v, stage_v, deg_sh):
    c = lax.axis_index("c")
    s = lax.axis_index("s")
    wid = c * NS + s
    pltpu.sync_copy(ones_hbm, ones_v)
    pltpu.sync_copy(zeros_hbm, stage_v)

    def zinit(j, carry):
        pltpu.sync_copy(stage_v, deg_sh.at[pl.ds(s * RP + j * RC, RC), :])
        return carry

    lax.fori_loop(0, RP // RC, zinit, 0)
    plsc.subcore_barrier()

    def body(i, carry):
        off = pl.multiple_of(wid * EPW + i * CE, 8)
        pltpu.sync_copy(dst_hbm.at[pl.ds(off, CE)], dst_v)
        pltpu.sync_copy(ones_v, deg_sh.at[dst_v], add=True)
        return carry

    lax.fori_loop(0, EPW // CE, body, 0)
    plsc.subcore_barrier()

    def cout(j, carry):
        r = s * RP + j * RC
        pltpu.sync_copy(deg_sh.at[pl.ds(r, RC), :], stage_v)
        pltpu.sync_copy(stage_v, out_hbm.at[c, pl.ds(r, RC), :])
        return carry

    lax.fori_loop(0, RP // RC, cout, 0)


def _scatter_body(ylo_hbm, yhi_hbm, src_hbm, dst_hbm, zeros_hbm, out_hbm,
                  src_v0, src_v1, dst_v0, dst_v1, rows_v0, rows_v1, stage_v,
                  agg_sh, sis0, sid0, sis1, sid1, sg0, sg1, ss0, ss1):
    c = lax.axis_index("c")
    s = lax.axis_index("s")
    pltpu.sync_copy(zeros_hbm, stage_v)

    def zinit(j, carry):
        pltpu.sync_copy(stage_v, agg_sh.at[pl.ds(s * RP + j * RC, RC), :])
        return carry

    lax.fori_loop(0, RP // RC, zinit, 0)
    plsc.subcore_barrier()

    base = s * EPT

    def fire_idx(i, sv, dv, sis, sid):
        off = pl.multiple_of(base + i * CE, 8)
        pltpu.async_copy(src_hbm.at[pl.ds(off, CE)], sv, sis)
        pltpu.async_copy(dst_hbm.at[pl.ds(off, CE)], dv, sid)

    def wait_idx(sv, dv, sis, sid):
        pltpu.make_async_copy(src_hbm.at[pl.ds(0, CE)], sv, sis).wait()
        pltpu.make_async_copy(dst_hbm.at[pl.ds(0, CE)], dv, sid).wait()

    def edge_loop(tbl_hbm):
        fire_idx(0, src_v0, dst_v0, sis0, sid0)

        def body(j, carry):
            @pl.when(j > 0)
            def _():
                pltpu.make_async_copy(rows_v1, agg_sh.at[dst_v1], ss1).wait()

            fire_idx(2 * j + 1, src_v1, dst_v1, sis1, sid1)
            wait_idx(src_v0, dst_v0, sis0, sid0)
            g0 = pltpu.async_copy(tbl_hbm.at[src_v0], rows_v0, sg0)
            wait_idx(src_v1, dst_v1, sis1, sid1)
            g1 = pltpu.async_copy(tbl_hbm.at[src_v1], rows_v1, sg1)
            g0.wait()
            sc0 = pltpu.async_copy(rows_v0, agg_sh.at[dst_v0], ss0, add=True)
            g1.wait()
            pltpu.async_copy(rows_v1, agg_sh.at[dst_v1], ss1, add=True)
            sc0.wait()

            @pl.when(j < NCH // 2 - 1)
            def _():
                fire_idx(2 * j + 2, src_v0, dst_v0, sis0, sid0)

            return carry

        lax.fori_loop(0, NCH // 2, body, 0)
        pltpu.make_async_copy(rows_v1, agg_sh.at[dst_v1], ss1).wait()

    pl.when(c == 0)(lambda: edge_loop(ylo_hbm))
    pl.when(c == 1)(lambda: edge_loop(yhi_hbm))
    plsc.subcore_barrier()

    def cout(j, carry):
        r = s * RP + j * RC
        pltpu.sync_copy(agg_sh.at[pl.ds(r, RC), :], stage_v)
        pltpu.sync_copy(stage_v, out_hbm.at[c, pl.ds(r, RC), :])
        return carry

    lax.fori_loop(0, RP // RC, cout, 0)


@functools.cache
def _build_sc_kernels():
    mesh = plsc.VectorSubcoreMesh(core_axis_name="c", subcore_axis_name="s",
                                  num_cores=NC, num_subcores=NS)
    deg = pl.kernel(
        _deg_body,
        out_type=jax.ShapeDtypeStruct((NC, NP, DW), jnp.float32),
        mesh=mesh,
        scratch_types=[
            pltpu.VMEM((CE,), jnp.int32),
            pltpu.VMEM((CE, DW), jnp.float32),
            pltpu.VMEM((RC, DW), jnp.float32),
            pltpu.VMEM_SHARED((NP, DW), jnp.float32),
        ],
    )
    scat = pl.kernel(
        _scatter_body,
        out_type=jax.ShapeDtypeStruct((NC, NP, CB), jnp.float32),
        mesh=mesh,
        scratch_types=[
            pltpu.VMEM((CE,), jnp.int32),
            pltpu.VMEM((CE,), jnp.int32),
            pltpu.VMEM((CE,), jnp.int32),
            pltpu.VMEM((CE,), jnp.int32),
            pltpu.VMEM((CE, CB), jnp.float32),
            pltpu.VMEM((CE, CB), jnp.float32),
            pltpu.VMEM((RC, CB), jnp.float32),
            pltpu.VMEM_SHARED((NP, CB), jnp.float32),
        ] + [pltpu.SemaphoreType.DMA] * 8,
    )
    return deg, scat


def _deg_kernel(dst, ones_dw, zeros_dw):
    return _build_sc_kernels()[0](dst, ones_dw, zeros_dw)


def _scatter_kernel(ylo, yhi, src, dst, zeros_cb):
    return _build_sc_kernels()[1](ylo, yhi, src, dst, zeros_cb)


_RB = 1000
_GRID = (N // _RB,)


def _row_spec(w):
    return pl.BlockSpec((_RB, w), lambda i: (i, 0))


def _part_spec(part, w):
    return pl.BlockSpec((1, _RB, w), lambda i, _p=part: (_p, i, 0))


def _full_spec(r, ccols):
    return pl.BlockSpec((r, ccols), lambda i: (0, 0))


def _mlp_body(x_ref, w1_ref, b1_ref, w2_ref, b2_ref, h_ref):
    h1 = jnp.dot(x_ref[...], w1_ref[...], preferred_element_type=jnp.float32)
    h1 = jnp.maximum(h1 + b1_ref[...], 0.0)
    h2 = jnp.dot(h1, w2_ref[...], preferred_element_type=jnp.float32)
    h_ref[...] = jnp.maximum(h2 + b2_ref[...], 0.0)


def _mlp_call(x, w1p, b1p, w2p, b2p):
    return pl.pallas_call(
        _mlp_body,
        grid=_GRID,
        in_specs=[_row_spec(F), _full_spec(F, HP), _full_spec(1, HP),
                  _full_spec(HP, HP), _full_spec(1, HP)],
        out_specs=_row_spec(HP),
        out_shape=jax.ShapeDtypeStruct((N, HP), jnp.float32),
    )(x, w1p, b1p, w2p, b2p)


def _split_y(fn, d, ylo_ref, yhi_ref):
    y = fn * d
    ylo_ref[...] = y[:, :CB]
    yhi_ref[...] = jnp.concatenate(
        [y[:, CB:], jnp.zeros((y.shape[0], CB - HI), jnp.float32)], axis=1)


def _dinv_body(d0_ref, d1_ref, h_ref, dinv_ref, ylo_ref, yhi_ref,
               acc1_ref, acc2_ref):
    deg = d0_ref[0] + d1_ref[0]
    dinv = lax.rsqrt(jnp.maximum(deg, 1.0))
    dinv_ref[...] = dinv
    h = h_ref[...]
    _split_y(h, dinv[:, 0:1], ylo_ref, yhi_ref)
    acc1_ref[...] = TH1[0] * h
    acc2_ref[...] = TH2[0] * h


def _dinv_call(degp, h):
    return pl.pallas_call(
        _dinv_body,
        grid=_GRID,
        in_specs=[_part_spec(0, DW), _part_spec(1, DW), _row_spec(HP)],
        out_specs=[_row_spec(DW), _row_spec(CB), _row_spec(CB),
                   _row_spec(HP), _row_spec(HP)],
        out_shape=[jax.ShapeDtypeStruct((N, DW), jnp.float32),
                   jax.ShapeDtypeStruct((N, CB), jnp.float32),
                   jax.ShapeDtypeStruct((N, CB), jnp.float32),
                   jax.ShapeDtypeStruct((N, HP), jnp.float32),
                   jax.ShapeDtypeStruct((N, HP), jnp.float32)],
    )(degp, degp, h)


def _combine_body(th_ref, feat_ref, a0_ref, a1_ref, dinv_ref, acc1_ref,
                  acc2_ref, featn_ref, ylo_ref, yhi_ref,
                  acc1o_ref, acc2o_ref):
    d = dinv_ref[:, 0:1]
    agg = jnp.concatenate([a0_ref[0], a1_ref[0][:, :HI]], axis=1)
    fn = feat_ref[...] - agg * d
    featn_ref[...] = fn
    _split_y(fn, d, ylo_ref, yhi_ref)
    acc1o_ref[...] = acc1_ref[...] + th_ref[0] * fn
    acc2o_ref[...] = acc2_ref[...] + th_ref[1] * fn


def _combine_call(th, feat, aggp, dinv, acc1, acc2):
    return pl.pallas_call(
        _combine_body,
        grid=_GRID,
        in_specs=[pl.BlockSpec(memory_space=pltpu.SMEM),
                  _row_spec(HP), _part_spec(0, CB), _part_spec(1, CB),
                  _row_spec(DW), _row_spec(HP), _row_spec(HP)],
        out_specs=[_row_spec(HP), _row_spec(CB), _row_spec(CB),
                   _row_spec(HP), _row_spec(HP)],
        out_shape=[jax.ShapeDtypeStruct((N, HP), jnp.float32),
                   jax.ShapeDtypeStruct((N, CB), jnp.float32),
                   jax.ShapeDtypeStruct((N, CB), jnp.float32),
                   jax.ShapeDtypeStruct((N, HP), jnp.float32),
                   jax.ShapeDtypeStruct((N, HP), jnp.float32)],
    )(th, feat, aggp, aggp, dinv, acc1, acc2)


def _head_body(h_ref, acc1_ref, acc2_ref, wa_ref, wb_ref, wc_ref,
               bm1_ref, wm2_ref, bm2_ref, out_ref):
    z = jnp.dot(h_ref[...], wa_ref[...], preferred_element_type=jnp.float32)
    z += jnp.dot(acc1_ref[...], wb_ref[...], preferred_element_type=jnp.float32)
    z += jnp.dot(acc2_ref[...], wc_ref[...], preferred_element_type=jnp.float32)
    z = jnp.maximum(z + bm1_ref[...], 0.0)
    out_ref[...] = (jnp.dot(z, wm2_ref[...], preferred_element_type=jnp.float32)
                    + bm2_ref[...])


def _head_call(h, acc1, acc2, wa, wb, wc, bm1p, wm2p, bm2):
    return pl.pallas_call(
        _head_body,
        grid=_GRID,
        in_specs=[_row_spec(HP)] * 3 + [_full_spec(HP, HP)] * 3
                 + [_full_spec(1, HP), _full_spec(HP, NCLS), _full_spec(1, NCLS)],
        out_specs=_row_spec(NCLS),
        out_shape=jax.ShapeDtypeStruct((N, NCLS), jnp.float32),
    )(h, acc1, acc2, wa, wb, wc, bm1p, wm2p, bm2)


def kernel(x, edge_index, W1, b1, W2, b2, Wm1, bm1, Wm2, bm2):
    src = edge_index[0]
    dst = edge_index[1]

    w1p = jnp.pad(W1, ((0, 0), (0, HP - H)))
    b1p = jnp.pad(b1, (0, HP - H)).reshape(1, HP)
    w2p = jnp.pad(W2, ((0, HP - H), (0, HP - H)))
    b2p = jnp.pad(b2, (0, HP - H)).reshape(1, HP)
    wa = jnp.pad(Wm1[0:H], ((0, HP - H), (0, HP - H)))
    wb = jnp.pad(Wm1[H:2 * H], ((0, HP - H), (0, HP - H)))
    wc = jnp.pad(Wm1[2 * H:3 * H], ((0, HP - H), (0, HP - H)))
    bm1p = jnp.pad(bm1, (0, HP - H)).reshape(1, HP)
    wm2p = jnp.pad(Wm2, ((0, HP - H), (0, 0)))
    bm2p = bm2.reshape(1, NCLS)

    ones_dw = jnp.ones((CE, DW), jnp.float32)
    zeros_dw = jnp.zeros((RC, DW), jnp.float32)
    zeros_cb = jnp.zeros((RC, CB), jnp.float32)

    h = _mlp_call(x, w1p, b1p, w2p, b2p)
    degp = _deg_kernel(dst, ones_dw, zeros_dw)
    dinv, ylo, yhi, acc1, acc2 = _dinv_call(degp, h)

    ths = jnp.array([[TH1[1], TH2[1]], [TH1[2], TH2[2]], [TH1[3], TH2[3]]],
                    jnp.float32)

    def step(carry, th):
        feat, ylo, yhi, acc1, acc2 = carry
        aggp = _scatter_kernel(ylo, yhi, src, dst, zeros_cb)
        feat, ylo, yhi, acc1, acc2 = _combine_call(th, feat, aggp, dinv,
                                                   acc1, acc2)
        return (feat, ylo, yhi, acc1, acc2), None

    (_, _, _, acc1, acc2), _ = lax.scan(step, (h, ylo, yhi, acc1, acc2), ths)

    return _head_call(h, acc1, acc2, wa, wb, wc, bm1p, wm2p, bm2p)


# speedup vs baseline: 5.5906x; 1.2554x over previous
import functools

import jax
import jax.numpy as jnp
from jax import lax
from jax.experimental import pallas as pl
from jax.experimental.pallas import tpu as pltpu
from jax.experimental.pallas import tpu_sc as plsc

N = 10000
E = 320000
F = 128
H = 153
HP = 160
CB = 128
HI = HP - CB
NCLS = 2

NC, NS = 2, 16
EPT = E // NS
CE = 80
NCH = EPT // CE
NP = 10240
RP = NP // NS
RC = 32
DW = 128
EPW = E // (NC * NS)

TH1 = (-0.018739098133068916, 0.22664318420656426,
       -1.1625027523916962, 1.3784681394089935)
TH2 = (0.11613730625866586, -0.9204508026677373,
       2.2984110493405274, -0.8451376850831508)


def _deg_body(dst_hbm, ones_hbm, zeros_hbm, out_hbm,
              dst_v, ones_v, stage_v, deg_sh):
    c = lax.axis_index("c")
    s = lax.axis_index("s")
    wid = c * NS + s
    pltpu.sync_copy(ones_hbm, ones_v)
    pltpu.sync_copy(zeros_hbm, stage_v)

    def zinit(j, carry):
        pltpu.sync_copy(stage_v, deg_sh.at[pl.ds(s * RP + j * RC, RC), :])
        return carry

    lax.fori_loop(0, RP // RC, zinit, 0)
    plsc.subcore_barrier()

    def body(i, carry):
        off = pl.multiple_of(wid * EPW + i * CE, 8)
        pltpu.sync_copy(dst_hbm.at[pl.ds(off, CE)], dst_v)
        pltpu.sync_copy(ones_v, deg_sh.at[dst_v], add=True)
        return carry

    lax.fori_loop(0, EPW // CE, body, 0)
    plsc.subcore_barrier()

    def cout(j, carry):
        r = s * RP + j * RC
        pltpu.sync_copy(deg_sh.at[pl.ds(r, RC), :], stage_v)
        pltpu.sync_copy(stage_v, out_hbm.at[c, pl.ds(r, RC), :])
        return carry

    lax.fori_loop(0, RP // RC, cout, 0)


def _scatter_body(ylo_hbm, yhi_hbm, src_hbm, dst_hbm, zeros_hbm, out_hbm,
                  *sc):
    src_vs, dst_vs, rows_vs = sc[0:4], sc[4:8], sc[8:12]
    stage_v, agg_sh = sc[12], sc[13]
    sis, sid, sg, ss = sc[14:18], sc[18:22], sc[22:26], sc[26:30]
    c = lax.axis_index("c")
    s = lax.axis_index("s")
    pltpu.sync_copy(zeros_hbm, stage_v)

    def zinit(j, carry):
        pltpu.sync_copy(stage_v, agg_sh.at[pl.ds(s * RP + j * RC, RC), :])
        return carry

    lax.fori_loop(0, RP // RC, zinit, 0)
    plsc.subcore_barrier()

    base = s * EPT

    def fire_idx(k, i):
        off = pl.multiple_of(base + i * CE, 8)
        pltpu.async_copy(src_hbm.at[pl.ds(off, CE)], src_vs[k], sis[k])
        pltpu.async_copy(dst_hbm.at[pl.ds(off, CE)], dst_vs[k], sid[k])

    def wait_idx(k):
        pltpu.make_async_copy(src_hbm.at[pl.ds(0, CE)], src_vs[k], sis[k]).wait()
        pltpu.make_async_copy(dst_hbm.at[pl.ds(0, CE)], dst_vs[k], sid[k]).wait()

    def drain_s(k):
        pltpu.make_async_copy(rows_vs[k], agg_sh.at[dst_vs[k]], ss[k]).wait()

    def edge_loop(tbl_hbm):
        fire_idx(0, 0)
        fire_idx(1, 1)

        def body(j, carry):
            g = [None] * 4
            for k in (0, 1):
                wait_idx(k)
                g[k] = pltpu.async_copy(tbl_hbm.at[src_vs[k]], rows_vs[k], sg[k])
            for k in (2, 3):
                pl.when(j > 0)(functools.partial(drain_s, k))
                fire_idx(k, 4 * j + k)
            for k in (0, 1):
                g[k].wait()
                pltpu.async_copy(rows_vs[k], agg_sh.at[dst_vs[k]], ss[k],
                                 add=True)
            for k in (2, 3):
                wait_idx(k)
                g[k] = pltpu.async_copy(tbl_hbm.at[src_vs[k]], rows_vs[k], sg[k])
            for k in (2, 3):
                g[k].wait()
                pltpu.async_copy(rows_vs[k], agg_sh.at[dst_vs[k]], ss[k],
                                 add=True)
            for k in (0, 1):
                drain_s(k)
                fire_idx(k, 4 * j + 4 + k)
            return carry

        nfull = NCH // 4
        lax.fori_loop(0, nfull, body, 0)
        for k in (2, 3):
            drain_s(k)
        g = [None, None]
        for k in (0, 1):
            wait_idx(k)
            g[k] = pltpu.async_copy(tbl_hbm.at[src_vs[k]], rows_vs[k], sg[k])
        for k in (0, 1):
            g[k].wait()
            pltpu.async_copy(rows_vs[k], agg_sh.at[dst_vs[k]], ss[k], add=True)
        for k in (0, 1):
            drain_s(k)

    pl.when(c == 0)(lambda: edge_loop(ylo_hbm))
    pl.when(c == 1)(lambda: edge_loop(yhi_hbm))
    plsc.subcore_barrier()

    def cout(j, carry):
        r = s * RP + j * RC
        pltpu.sync_copy(agg_sh.at[pl.ds(r, RC), :], stage_v)
        pltpu.sync_copy(stage_v, out_hbm.at[c, pl.ds(r, RC), :])
        return carry

    lax.fori_loop(0, RP // RC, cout, 0)


@functools.cache
def _build_sc_kernels():
    mesh = plsc.VectorSubcoreMesh(core_axis_name="c", subcore_axis_name="s",
                                  num_cores=NC, num_subcores=NS)
    deg = pl.kernel(
        _deg_body,
        out_type=jax.ShapeDtypeStruct((NC, NP, DW), jnp.float32),
        mesh=mesh,
        scratch_types=[
            pltpu.VMEM((CE,), jnp.int32),
            pltpu.VMEM((CE, DW), jnp.float32),
            pltpu.VMEM((RC, DW), jnp.float32),
            pltpu.VMEM_SHARED((NP, DW), jnp.float32),
        ],
    )
    scat = pl.kernel(
        _scatter_body,
        out_type=jax.ShapeDtypeStruct((NC, NP, CB), jnp.float32),
        mesh=mesh,
        scratch_types=[pltpu.VMEM((CE,), jnp.int32)] * 8
                      + [pltpu.VMEM((CE, CB), jnp.float32)] * 4
                      + [pltpu.VMEM((RC, CB), jnp.float32),
                         pltpu.VMEM_SHARED((NP, CB), jnp.float32)]
                      + [pltpu.SemaphoreType.DMA] * 16,
    )
    return deg, scat


def _deg_kernel(dst, ones_dw, zeros_dw):
    return _build_sc_kernels()[0](dst, ones_dw, zeros_dw)


def _scatter_kernel(ylo, yhi, src, dst, zeros_cb):
    return _build_sc_kernels()[1](ylo, yhi, src, dst, zeros_cb)


_RB = 1000
_GRID = (N // _RB,)


def _row_spec(w):
    return pl.BlockSpec((_RB, w), lambda i: (i, 0))


def _part_spec(part, w):
    return pl.BlockSpec((1, _RB, w), lambda i, _p=part: (_p, i, 0))


def _full_spec(r, ccols):
    return pl.BlockSpec((r, ccols), lambda i: (0, 0))


def _mlp_body(x_ref, w1_ref, b1_ref, w2_ref, b2_ref, h_ref):
    h1 = jnp.dot(x_ref[...], w1_ref[...], preferred_element_type=jnp.float32)
    h1 = jnp.maximum(h1 + b1_ref[...], 0.0)
    h2 = jnp.dot(h1, w2_ref[...], preferred_element_type=jnp.float32)
    h_ref[...] = jnp.maximum(h2 + b2_ref[...], 0.0)


def _mlp_call(x, w1p, b1p, w2p, b2p):
    return pl.pallas_call(
        _mlp_body,
        grid=_GRID,
        in_specs=[_row_spec(F), _full_spec(F, HP), _full_spec(1, HP),
                  _full_spec(HP, HP), _full_spec(1, HP)],
        out_specs=_row_spec(HP),
        out_shape=jax.ShapeDtypeStruct((N, HP), jnp.float32),
    )(x, w1p, b1p, w2p, b2p)


def _split_y(fn, d, ylo_ref, yhi_ref):
    y = fn * d
    ylo_ref[...] = y[:, :CB]
    yhi_ref[...] = jnp.concatenate(
        [y[:, CB:], jnp.zeros((y.shape[0], CB - HI), jnp.float32)], axis=1)


def _dinv_body(d0_ref, d1_ref, h_ref, dinv_ref, ylo_ref, yhi_ref,
               acc1_ref, acc2_ref):
    deg = d0_ref[0] + d1_ref[0]
    dinv = lax.rsqrt(jnp.maximum(deg, 1.0))
    dinv_ref[...] = dinv
    h = h_ref[...]
    _split_y(h, dinv[:, 0:1], ylo_ref, yhi_ref)
    acc1_ref[...] = TH1[0] * h
    acc2_ref[...] = TH2[0] * h


def _dinv_call(degp, h):
    return pl.pallas_call(
        _dinv_body,
        grid=_GRID,
        in_specs=[_part_spec(0, DW), _part_spec(1, DW), _row_spec(HP)],
        out_specs=[_row_spec(DW), _row_spec(CB), _row_spec(CB),
                   _row_spec(HP), _row_spec(HP)],
        out_shape=[jax.ShapeDtypeStruct((N, DW), jnp.float32),
                   jax.ShapeDtypeStruct((N, CB), jnp.float32),
                   jax.ShapeDtypeStruct((N, CB), jnp.float32),
                   jax.ShapeDtypeStruct((N, HP), jnp.float32),
                   jax.ShapeDtypeStruct((N, HP), jnp.float32)],
    )(degp, degp, h)


def _combine_body(th_ref, feat_ref, a0_ref, a1_ref, dinv_ref, acc1_ref,
                  acc2_ref, featn_ref, ylo_ref, yhi_ref,
                  acc1o_ref, acc2o_ref):
    d = dinv_ref[:, 0:1]
    agg = jnp.concatenate([a0_ref[0], a1_ref[0][:, :HI]], axis=1)
    fn = feat_ref[...] - agg * d
    featn_ref[...] = fn
    _split_y(fn, d, ylo_ref, yhi_ref)
    acc1o_ref[...] = acc1_ref[...] + th_ref[0] * fn
    acc2o_ref[...] = acc2_ref[...] + th_ref[1] * fn


def _combine_call(th, feat, aggp, dinv, acc1, acc2):
    return pl.pallas_call(
        _combine_body,
        grid=_GRID,
        in_specs=[pl.BlockSpec(memory_space=pltpu.SMEM),
                  _row_spec(HP), _part_spec(0, CB), _part_spec(1, CB),
                  _row_spec(DW), _row_spec(HP), _row_spec(HP)],
        out_specs=[_row_spec(HP), _row_spec(CB), _row_spec(CB),
                   _row_spec(HP), _row_spec(HP)],
        out_shape=[jax.ShapeDtypeStruct((N, HP), jnp.float32),
                   jax.ShapeDtypeStruct((N, CB), jnp.float32),
                   jax.ShapeDtypeStruct((N, CB), jnp.float32),
                   jax.ShapeDtypeStruct((N, HP), jnp.float32),
                   jax.ShapeDtypeStruct((N, HP), jnp.float32)],
    )(th, feat, aggp, aggp, dinv, acc1, acc2)


def _head_body(h_ref, acc1_ref, acc2_ref, wa_ref, wb_ref, wc_ref,
               bm1_ref, wm2_ref, bm2_ref, out_ref):
    z = jnp.dot(h_ref[...], wa_ref[...], preferred_element_type=jnp.float32)
    z += jnp.dot(acc1_ref[...], wb_ref[...], preferred_element_type=jnp.float32)
    z += jnp.dot(acc2_ref[...], wc_ref[...], preferred_element_type=jnp.float32)
    z = jnp.maximum(z + bm1_ref[...], 0.0)
    out_ref[...] = (jnp.dot(z, wm2_ref[...], preferred_element_type=jnp.float32)
                    + bm2_ref[...])


def _head_call(h, acc1, acc2, wa, wb, wc, bm1p, wm2p, bm2):
    return pl.pallas_call(
        _head_body,
        grid=_GRID,
        in_specs=[_row_spec(HP)] * 3 + [_full_spec(HP, HP)] * 3
                 + [_full_spec(1, HP), _full_spec(HP, NCLS), _full_spec(1, NCLS)],
        out_specs=_row_spec(NCLS),
        out_shape=jax.ShapeDtypeStruct((N, NCLS), jnp.float32),
    )(h, acc1, acc2, wa, wb, wc, bm1p, wm2p, bm2)


def kernel(x, edge_index, W1, b1, W2, b2, Wm1, bm1, Wm2, bm2):
    src = edge_index[0]
    dst = edge_index[1]

    w1p = jnp.pad(W1, ((0, 0), (0, HP - H)))
    b1p = jnp.pad(b1, (0, HP - H)).reshape(1, HP)
    w2p = jnp.pad(W2, ((0, HP - H), (0, HP - H)))
    b2p = jnp.pad(b2, (0, HP - H)).reshape(1, HP)
    wa = jnp.pad(Wm1[0:H], ((0, HP - H), (0, HP - H)))
    wb = jnp.pad(Wm1[H:2 * H], ((0, HP - H), (0, HP - H)))
    wc = jnp.pad(Wm1[2 * H:3 * H], ((0, HP - H), (0, HP - H)))
    bm1p = jnp.pad(bm1, (0, HP - H)).reshape(1, HP)
    wm2p = jnp.pad(Wm2, ((0, HP - H), (0, 0)))
    bm2p = bm2.reshape(1, NCLS)

    ones_dw = jnp.ones((CE, DW), jnp.float32)
    zeros_dw = jnp.zeros((RC, DW), jnp.float32)
    zeros_cb = jnp.zeros((RC, CB), jnp.float32)

    h = _mlp_call(x, w1p, b1p, w2p, b2p)
    degp = _deg_kernel(dst, ones_dw, zeros_dw)
    dinv, ylo, yhi, acc1, acc2 = _dinv_call(degp, h)

    ths = jnp.array([[TH1[1], TH2[1]], [TH1[2], TH2[2]], [TH1[3], TH2[3]]],
                    jnp.float32)

    def step(carry, th):
        feat, ylo, yhi, acc1, acc2 = carry
        aggp = _scatter_kernel(ylo, yhi, src, dst, zeros_cb)
        feat, ylo, yhi, acc1, acc2 = _combine_call(th, feat, aggp, dinv,
                                                   acc1, acc2)
        return (feat, ylo, yhi, acc1, acc2), None

    (_, _, _, acc1, acc2), _ = lax.scan(step, (h, ylo, yhi, acc1, acc2), ths)

    return _head_call(h, acc1, acc2, wa, wb, wc, bm1p, wm2p, bm2p)


# speedup vs baseline: 5.8488x; 1.0462x over previous
import functools

import jax
import jax.numpy as jnp
from jax import lax
from jax.experimental import pallas as pl
from jax.experimental.pallas import tpu as pltpu
from jax.experimental.pallas import tpu_sc as plsc

N = 10000
E = 320000
F = 128
H = 153
HP = 160
CB = 128
HI = HP - CB
NCLS = 2

NC, NS = 2, 16
EPT = E // NS
CE = 80
NCH = EPT // CE
NP = 10240
RP = NP // NS
RC = 32
DW = 128
EPW = E // (NC * NS)
DVW = 8

TH1 = (-0.018739098133068916, 0.22664318420656426,
       -1.1625027523916962, 1.3784681394089935)
TH2 = (0.11613730625866586, -0.9204508026677373,
       2.2984110493405274, -0.8451376850831508)


def _deg_body(dst_hbm, ones_hbm, zeros_hbm, out_hbm,
              dst_v, ones_v, stage_v, deg_sh):
    c = lax.axis_index("c")
    s = lax.axis_index("s")
    wid = c * NS + s
    pltpu.sync_copy(ones_hbm, ones_v)
    pltpu.sync_copy(zeros_hbm, stage_v)

    def zinit(j, carry):
        pltpu.sync_copy(stage_v, deg_sh.at[pl.ds(s * RP + j * RC, RC), :])
        return carry

    lax.fori_loop(0, RP // RC, zinit, 0)
    plsc.subcore_barrier()

    def body(i, carry):
        off = pl.multiple_of(wid * EPW + i * CE, 8)
        pltpu.sync_copy(dst_hbm.at[pl.ds(off, CE)], dst_v)
        pltpu.sync_copy(ones_v, deg_sh.at[dst_v], add=True)
        return carry

    lax.fori_loop(0, EPW // CE, body, 0)
    plsc.subcore_barrier()

    def cout(j, carry):
        r = s * RP + j * RC
        pltpu.sync_copy(deg_sh.at[pl.ds(r, RC), :], stage_v)
        pltpu.sync_copy(stage_v, out_hbm.at[c, pl.ds(r, RC), :])
        return carry

    lax.fori_loop(0, RP // RC, cout, 0)


def _scatter_body(ylo_hbm, yhi_hbm, src_hbm, dst_hbm, zeros_hbm, out_hbm,
                  *sc):
    src_vs, dst_vs, rows_vs = sc[0:4], sc[4:8], sc[8:12]
    stage_v, agg_sh = sc[12], sc[13]
    sis, sid, sg, ss = sc[14:18], sc[18:22], sc[22:26], sc[26:30]
    c = lax.axis_index("c")
    s = lax.axis_index("s")
    pltpu.sync_copy(zeros_hbm, stage_v)

    def zinit(j, carry):
        pltpu.sync_copy(stage_v, agg_sh.at[pl.ds(s * RP + j * RC, RC), :])
        return carry

    lax.fori_loop(0, RP // RC, zinit, 0)
    plsc.subcore_barrier()

    base = s * EPT

    def fire_idx(k, i):
        off = pl.multiple_of(base + i * CE, 8)
        pltpu.async_copy(src_hbm.at[pl.ds(off, CE)], src_vs[k], sis[k])
        pltpu.async_copy(dst_hbm.at[pl.ds(off, CE)], dst_vs[k], sid[k])

    def wait_idx(k):
        pltpu.make_async_copy(src_hbm.at[pl.ds(0, CE)], src_vs[k], sis[k]).wait()
        pltpu.make_async_copy(dst_hbm.at[pl.ds(0, CE)], dst_vs[k], sid[k]).wait()

    def drain_s(k):
        pltpu.make_async_copy(rows_vs[k], agg_sh.at[dst_vs[k]], ss[k]).wait()

    def edge_loop(tbl_hbm):
        fire_idx(0, 0)
        fire_idx(1, 1)

        def body(j, carry):
            g = [None] * 4
            for k in (0, 1):
                wait_idx(k)
                g[k] = pltpu.async_copy(tbl_hbm.at[src_vs[k]], rows_vs[k], sg[k])
            for k in (2, 3):
                pl.when(j > 0)(functools.partial(drain_s, k))
                fire_idx(k, 4 * j + k)
            for k in (0, 1):
                g[k].wait()
                pltpu.async_copy(rows_vs[k], agg_sh.at[dst_vs[k]], ss[k],
                                 add=True)
            for k in (2, 3):
                wait_idx(k)
                g[k] = pltpu.async_copy(tbl_hbm.at[src_vs[k]], rows_vs[k], sg[k])
            for k in (2, 3):
                g[k].wait()
                pltpu.async_copy(rows_vs[k], agg_sh.at[dst_vs[k]], ss[k],
                                 add=True)
            for k in (0, 1):
                drain_s(k)
                fire_idx(k, 4 * j + 4 + k)
            return carry

        nfull = NCH // 4
        lax.fori_loop(0, nfull, body, 0)
        for k in (2, 3):
            drain_s(k)
        g = [None, None]
        for k in (0, 1):
            wait_idx(k)
            g[k] = pltpu.async_copy(tbl_hbm.at[src_vs[k]], rows_vs[k], sg[k])
        for k in (0, 1):
            g[k].wait()
            pltpu.async_copy(rows_vs[k], agg_sh.at[dst_vs[k]], ss[k], add=True)
        for k in (0, 1):
            drain_s(k)

    pl.when(c == 0)(lambda: edge_loop(ylo_hbm))
    pl.when(c == 1)(lambda: edge_loop(yhi_hbm))
    plsc.subcore_barrier()

    def cout(j, carry):
        r = s * RP + j * RC
        pltpu.sync_copy(agg_sh.at[pl.ds(r, RC), :], stage_v)
        pltpu.sync_copy(stage_v, out_hbm.at[c, pl.ds(r, RC), :])
        return carry

    lax.fori_loop(0, RP // RC, cout, 0)


@functools.cache
def _build_sc_kernels():
    mesh = plsc.VectorSubcoreMesh(core_axis_name="c", subcore_axis_name="s",
                                  num_cores=NC, num_subcores=NS)
    deg = pl.kernel(
        _deg_body,
        out_type=jax.ShapeDtypeStruct((NC, NP, DW), jnp.float32),
        mesh=mesh,
        scratch_types=[
            pltpu.VMEM((CE,), jnp.int32),
            pltpu.VMEM((CE, DW), jnp.float32),
            pltpu.VMEM((RC, DW), jnp.float32),
            pltpu.VMEM_SHARED((NP, DW), jnp.float32),
        ],
    )
    scat = pl.kernel(
        _scatter_body,
        out_type=jax.ShapeDtypeStruct((NC, NP, CB), jnp.float32),
        mesh=mesh,
        scratch_types=[pltpu.VMEM((CE,), jnp.int32)] * 8
                      + [pltpu.VMEM((CE, CB), jnp.float32)] * 4
                      + [pltpu.VMEM((RC, CB), jnp.float32),
                         pltpu.VMEM_SHARED((NP, CB), jnp.float32)]
                      + [pltpu.SemaphoreType.DMA] * 16,
    )
    return deg, scat


def _deg_kernel(dst, ones_dw, zeros_dw):
    return _build_sc_kernels()[0](dst, ones_dw, zeros_dw)


def _scatter_kernel(ylo, yhi, src, dst, zeros_cb):
    return _build_sc_kernels()[1](ylo, yhi, src, dst, zeros_cb)


_RB = 1000
_GRID = (N // _RB,)


def _row_spec(w):
    return pl.BlockSpec((_RB, w), lambda i: (i, 0))


def _part_spec(part, w):
    return pl.BlockSpec((1, _RB, w), lambda i, _p=part: (_p, i, 0))


def _full_spec(r, ccols):
    return pl.BlockSpec((r, ccols), lambda i: (0, 0))


def _mlp_body(x_ref, w1_ref, b1_ref, w2_ref, b2_ref, h_ref):
    h1 = jnp.dot(x_ref[...], w1_ref[...], preferred_element_type=jnp.float32)
    h1 = jnp.maximum(h1 + b1_ref[...], 0.0)
    h2 = jnp.dot(h1, w2_ref[...], preferred_element_type=jnp.float32)
    h_ref[...] = jnp.maximum(h2 + b2_ref[...], 0.0)


def _mlp_call(x, w1p, b1p, w2p, b2p):
    return pl.pallas_call(
        _mlp_body,
        grid=_GRID,
        in_specs=[_row_spec(F), _full_spec(F, HP), _full_spec(1, HP),
                  _full_spec(HP, HP), _full_spec(1, HP)],
        out_specs=_row_spec(HP),
        out_shape=jax.ShapeDtypeStruct((N, HP), jnp.float32),
    )(x, w1p, b1p, w2p, b2p)


def _split_y(fn, d, ylo_ref, yhi_ref):
    y = fn * d
    ylo_ref[...] = y[:, :CB]
    yhi_ref[...] = jnp.concatenate(
        [y[:, CB:], jnp.zeros((y.shape[0], CB - HI), jnp.float32)], axis=1)


def _dinv_body(d0_ref, d1_ref, h_ref, dinv_ref, ylo_ref, yhi_ref,
               acc1_ref, acc2_ref):
    deg = d0_ref[0] + d1_ref[0]
    dinv = lax.rsqrt(jnp.maximum(deg, 1.0))
    dinv_ref[...] = dinv[:, :DVW]
    h = h_ref[...]
    _split_y(h, dinv[:, 0:1], ylo_ref, yhi_ref)
    acc1_ref[...] = TH1[0] * h
    acc2_ref[...] = TH2[0] * h


def _dinv_call(degp, h):
    return pl.pallas_call(
        _dinv_body,
        grid=_GRID,
        in_specs=[_part_spec(0, DW), _part_spec(1, DW), _row_spec(HP)],
        out_specs=[_row_spec(DVW), _row_spec(CB), _row_spec(CB),
                   _row_spec(HP), _row_spec(HP)],
        out_shape=[jax.ShapeDtypeStruct((N, DVW), jnp.float32),
                   jax.ShapeDtypeStruct((N, CB), jnp.float32),
                   jax.ShapeDtypeStruct((N, CB), jnp.float32),
                   jax.ShapeDtypeStruct((N, HP), jnp.float32),
                   jax.ShapeDtypeStruct((N, HP), jnp.float32)],
    )(degp, degp, h)


def _agg_of(a0_ref, a1_ref):
    return jnp.concatenate([a0_ref[0], a1_ref[0][:, :HI]], axis=1)


def _ynext_body(feat_ref, a0_ref, a1_ref, dinv_ref,
                featn_ref, ylo_ref, yhi_ref):
    d = dinv_ref[:, 0:1]
    fn = feat_ref[...] - _agg_of(a0_ref, a1_ref) * d
    featn_ref[...] = fn
    _split_y(fn, d, ylo_ref, yhi_ref)


def _ynext_call(feat, aggp, dinv):
    return pl.pallas_call(
        _ynext_body,
        grid=_GRID,
        in_specs=[_row_spec(HP), _part_spec(0, CB), _part_spec(1, CB),
                  _row_spec(DVW)],
        out_specs=[_row_spec(HP), _row_spec(CB), _row_spec(CB)],
        out_shape=[jax.ShapeDtypeStruct((N, HP), jnp.float32),
                   jax.ShapeDtypeStruct((N, CB), jnp.float32),
                   jax.ShapeDtypeStruct((N, CB), jnp.float32)],
    )(feat, aggp, aggp, dinv)


def _accum_body(k, featn_ref, acc1_ref, acc2_ref, acc1o_ref, acc2o_ref):
    fn = featn_ref[...]
    acc1o_ref[...] = acc1_ref[...] + TH1[k] * fn
    acc2o_ref[...] = acc2_ref[...] + TH2[k] * fn


def _accum_call(k, featn, acc1, acc2):
    return pl.pallas_call(
        functools.partial(_accum_body, k),
        grid=_GRID,
        in_specs=[_row_spec(HP)] * 3,
        out_specs=[_row_spec(HP)] * 2,
        out_shape=[jax.ShapeDtypeStruct((N, HP), jnp.float32)] * 2,
    )(featn, acc1, acc2)


def _accum_final_body(k, feat_ref, a0_ref, a1_ref, dinv_ref, acc1_ref,
                      acc2_ref, acc1o_ref, acc2o_ref):
    d = dinv_ref[:, 0:1]
    fn = feat_ref[...] - _agg_of(a0_ref, a1_ref) * d
    acc1o_ref[...] = acc1_ref[...] + TH1[k] * fn
    acc2o_ref[...] = acc2_ref[...] + TH2[k] * fn


def _accum_final_call(k, feat, aggp, dinv, acc1, acc2):
    return pl.pallas_call(
        functools.partial(_accum_final_body, k),
        grid=_GRID,
        in_specs=[_row_spec(HP), _part_spec(0, CB), _part_spec(1, CB),
                  _row_spec(DVW), _row_spec(HP), _row_spec(HP)],
        out_specs=[_row_spec(HP)] * 2,
        out_shape=[jax.ShapeDtypeStruct((N, HP), jnp.float32)] * 2,
    )(feat, aggp, aggp, dinv, acc1, acc2)


def _head_body(h_ref, acc1_ref, acc2_ref, wa_ref, wb_ref, wc_ref,
               bm1_ref, wm2_ref, bm2_ref, out_ref):
    z = jnp.dot(h_ref[...], wa_ref[...], preferred_element_type=jnp.float32)
    z += jnp.dot(acc1_ref[...], wb_ref[...], preferred_element_type=jnp.float32)
    z += jnp.dot(acc2_ref[...], wc_ref[...], preferred_element_type=jnp.float32)
    z = jnp.maximum(z + bm1_ref[...], 0.0)
    out_ref[...] = (jnp.dot(z, wm2_ref[...], preferred_element_type=jnp.float32)
                    + bm2_ref[...])


def _head_call(h, acc1, acc2, wa, wb, wc, bm1p, wm2p, bm2):
    return pl.pallas_call(
        _head_body,
        grid=_GRID,
        in_specs=[_row_spec(HP)] * 3 + [_full_spec(HP, HP)] * 3
                 + [_full_spec(1, HP), _full_spec(HP, NCLS), _full_spec(1, NCLS)],
        out_specs=_row_spec(NCLS),
        out_shape=jax.ShapeDtypeStruct((N, NCLS), jnp.float32),
    )(h, acc1, acc2, wa, wb, wc, bm1p, wm2p, bm2)


def kernel(x, edge_index, W1, b1, W2, b2, Wm1, bm1, Wm2, bm2):
    src = edge_index[0]
    dst = edge_index[1]

    w1p = jnp.pad(W1, ((0, 0), (0, HP - H)))
    b1p = jnp.pad(b1, (0, HP - H)).reshape(1, HP)
    w2p = jnp.pad(W2, ((0, HP - H), (0, HP - H)))
    b2p = jnp.pad(b2, (0, HP - H)).reshape(1, HP)
    wa = jnp.pad(Wm1[0:H], ((0, HP - H), (0, HP - H)))
    wb = jnp.pad(Wm1[H:2 * H], ((0, HP - H), (0, HP - H)))
    wc = jnp.pad(Wm1[2 * H:3 * H], ((0, HP - H), (0, HP - H)))
    bm1p = jnp.pad(bm1, (0, HP - H)).reshape(1, HP)
    wm2p = jnp.pad(Wm2, ((0, HP - H), (0, 0)))
    bm2p = bm2.reshape(1, NCLS)

    ones_dw = jnp.ones((CE, DW), jnp.float32)
    zeros_dw = jnp.zeros((RC, DW), jnp.float32)
    zeros_cb = jnp.zeros((RC, CB), jnp.float32)

    h = _mlp_call(x, w1p, b1p, w2p, b2p)
    degp = _deg_kernel(dst, ones_dw, zeros_dw)
    dinv, ylo, yhi, acc1, acc2 = _dinv_call(degp, h)

    aggp = _scatter_kernel(ylo, yhi, src, dst, zeros_cb)
    featn, ylo, yhi = _ynext_call(h, aggp, dinv)
    aggp = _scatter_kernel(ylo, yhi, src, dst, zeros_cb)
    acc1, acc2 = _accum_call(1, featn, acc1, acc2)
    featn, ylo, yhi = _ynext_call(featn, aggp, dinv)
    aggp = _scatter_kernel(ylo, yhi, src, dst, zeros_cb)
    acc1, acc2 = _accum_call(2, featn, acc1, acc2)
    acc1, acc2 = _accum_final_call(3, featn, aggp, dinv, acc1, acc2)

    return _head_call(h, acc1, acc2, wa, wb, wc, bm1p, wm2p, bm2p)
